# Initial kernel scaffold; baseline (speedup 1.0000x reference)
import jax
import jax.numpy as jnp
from jax.experimental import pallas as pl


def kernel(embeddings, nodes):
    raise NotImplementedError("write your pallas kernel here")


# speedup vs baseline: 4.8960x; 4.8960x over previous
import jax
import jax.numpy as jnp
from jax.experimental import pallas as pl
from jax.experimental.pallas import tpu as pltpu

_G = 16
_N = _G * _G
_D = 128
_B = 4
_ITEMS = 256
_EPOCHS = 3
_LR = 0.01


def _som_body(emb_ref, nodes_ref, out_ref, grid_ref, sqn_ref):
    nodes = nodes_ref[...]
    grid_ref[...] = jnp.broadcast_to(nodes[None], (_B, _N, _D))
    sqn_ref[...] = jnp.broadcast_to(
        jnp.sum(nodes * nodes, axis=1)[None], (_B, _N))

    iota_n = jax.lax.broadcasted_iota(jnp.int32, (_B, _N), 1)
    c = jnp.float32(_LR)
    a2 = jnp.float32((1.0 - _LR) ** 2)
    ab = jnp.float32(2.0 * _LR * (1.0 - _LR))
    b2 = jnp.float32(_LR * _LR)

    def step(s, carry):
        t = jax.lax.rem(s, _ITEMS)
        x = emb_ref[t]
        g = grid_ref[...]
        sqn = sqn_ref[...]
        dots = jnp.sum(g * x[:, None, :], axis=2)
        score = sqn - 2.0 * dots
        m = jnp.min(score, axis=1, keepdims=True)
        bmu = jnp.min(jnp.where(score == m, iota_n, _N),
                      axis=1, keepdims=True)
        bx = bmu // _G
        by = jax.lax.rem(bmu, _G)
        mask = jnp.zeros((_B, _N), dtype=jnp.bool_)
        for dx, dy in ((0, 1), (1, 0), (0, -1), (-1, 0)):
            nx = bx + dx
            ny = by + dy
            valid = (nx >= 0) & (nx < _G) & (ny >= 0) & (ny < _G)
            nidx = jnp.clip(nx, 0, _G - 1) * _G + jnp.clip(ny, 0, _G - 1)
            mask = mask | (valid & (iota_n == nidx))
        w = jnp.where(mask, c, jnp.float32(0.0))
        grid_ref[...] = g + w[:, :, None] * (x[:, None, :] - g)
        xx = jnp.sum(x * x, axis=1)[:, None]
        sqn_ref[...] = jnp.where(mask, a2 * sqn + ab * dots + b2 * xx, sqn)
        return carry

    jax.lax.fori_loop(0, _EPOCHS * _ITEMS, step, 0)
    out_ref[...] = jnp.sum(grid_ref[...], axis=1)


def kernel(embeddings, nodes):
    emb_t = jnp.transpose(embeddings, (1, 0, 2))
    nodes_flat = nodes.reshape(_N, _D)
    return pl.pallas_call(
        _som_body,
        out_shape=jax.ShapeDtypeStruct((_B, _D), jnp.float32),
        scratch_shapes=[
            pltpu.VMEM((_B, _N, _D), jnp.float32),
            pltpu.VMEM((_B, _N), jnp.float32),
        ],
    )(emb_t, nodes_flat)


# speedup vs baseline: 5.3361x; 1.0899x over previous
import functools

import jax
import jax.numpy as jnp
from jax import lax
from jax.experimental import pallas as pl
from jax.experimental.pallas import tpu as pltpu
from jax.experimental.pallas import tpu_sc as plsc

_G = 16
_N = 256
_D = 128
_B = 4
_ITEMS = 256
_EPOCHS = 3
_LR = 0.01
_STEPS = _EPOCHS * _ITEMS


def _gram_body(emb_ref, nodes_ref, d0_ref, xx_ref, sqn0_ref):
    nodes = nodes_ref[...]
    sqn0_ref[...] = jnp.sum(nodes * nodes, axis=1, keepdims=True)
    for b in range(_B):
        xb = emb_ref[b]
        d0_ref[b, :, :] = jax.lax.dot_general(
            nodes, xb, (((1,), (1,)), ((), ())),
            preferred_element_type=jnp.float32)
        xx_ref[b, :, :] = jax.lax.dot_general(
            xb, xb, (((1,), (1,)), ((), ())),
            preferred_element_type=jnp.float32)


def _gram(embeddings, nodes_flat):
    return pl.pallas_call(
        _gram_body,
        out_shape=(
            jax.ShapeDtypeStruct((_B, _N, _ITEMS), jnp.float32),
            jax.ShapeDtypeStruct((_B, _ITEMS, _ITEMS), jnp.float32),
            jax.ShapeDtypeStruct((_N, 1), jnp.float32),
        ),
    )(embeddings, nodes_flat)


_mesh = plsc.VectorSubcoreMesh(core_axis_name="c", subcore_axis_name="s")


@functools.partial(
    pl.kernel,
    out_type=jax.ShapeDtypeStruct((_B, _D), jnp.float32),
    mesh=_mesh,
    scratch_types=[
        pltpu.VMEM((_N, _ITEMS), jnp.float32),
        pltpu.VMEM((_N * _D,), jnp.float32),
        pltpu.VMEM((16, 16), jnp.float32),
        pltpu.VMEM((2, _ITEMS), jnp.float32),
        pltpu.VMEM((2, _D), jnp.float32),
        pltpu.VMEM((_D,), jnp.float32),
        pltpu.SemaphoreType.DMA,
        pltpu.SemaphoreType.DMA,
        pltpu.SemaphoreType.DMA,
    ],
    compiler_params=pltpu.CompilerParams(needs_layout_passes=False),
)
def _som_sc(emb_hbm, nodes_hbm, sqn0_hbm, d0_hbm, xx_hbm, out_hbm,
            d_v, g_v, sqn_v, xx_v, x_v, out_v, sem_xx, sem_x, sem_p):
    wid = lax.axis_index("s") * 2 + lax.axis_index("c")

    @pl.when(wid < _B)
    def _body():
        b = wid
        iota = lax.iota(jnp.int32, 16)

        cp1 = pltpu.async_copy(d0_hbm.at[b], d_v, sem_p)
        cp2 = pltpu.async_copy(nodes_hbm, g_v, sem_p)
        cp3 = pltpu.async_copy(sqn0_hbm, sqn_v, sem_p)
        pltpu.async_copy(xx_hbm.at[b, 0], xx_v.at[0], sem_xx)
        pltpu.async_copy(emb_hbm.at[b, 0], x_v.at[0], sem_x)
        cp1.wait()
        cp2.wait()
        cp3.wait()

        one = jnp.int32(1)
        zero = jnp.int32(0)
        dxv = (jnp.where(iota == 1, one, zero)
               - jnp.where(iota == 3, one, zero))
        dyv = (jnp.where(iota == 0, one, zero)
               - jnp.where(iota == 2, one, zero))
        lane_lt4 = iota < 4

        def step(s, carry):
            t = s & (_ITEMS - 1)
            slot = s & 1
            nslot = 1 - slot
            tn = (s + 1) & (_ITEMS - 1)
            tvec = jnp.full((16,), t, jnp.int32)
            pltpu.make_async_copy(xx_hbm.at[b, t], xx_v.at[slot],
                                  sem_xx).wait()
            pltpu.make_async_copy(emb_hbm.at[b, t], x_v.at[slot],
                                  sem_x).wait()
            pltpu.async_copy(xx_hbm.at[b, tn], xx_v.at[nslot], sem_xx)
            pltpu.async_copy(emb_hbm.at[b, tn], x_v.at[nslot], sem_x)

            scores = []
            macc = None
            for c in range(16):
                dcol = plsc.load_gather(d_v, [iota + (c * 16), tvec])
                sc = sqn_v[c, :] - 2.0 * dcol
                scores.append(sc)
                macc = sc if macc is None else jnp.minimum(macc, sc)
            m = jnp.min(macc)
            cacc = None
            for c in range(16):
                cand = jnp.where(scores[c] == m, iota + (c * 16),
                                 jnp.int32(_N))
                cacc = cand if cacc is None else jnp.minimum(cacc, cand)
            bmu = jnp.min(cacc)

            bx = bmu >> 4
            by = bmu & 15
            nxv = bx + dxv
            nyv = by + dyv
            validv = (lane_lt4 & (nxv >= 0) & (nxv < _G)
                      & (nyv >= 0) & (nyv < _G))
            rv = (jnp.clip(nxv, 0, _G - 1) * _G
                  + jnp.clip(nyv, 0, _G - 1))
            cvec = jnp.where(validv, jnp.float32(_LR), jnp.float32(0.0))

            drow = plsc.load_gather(d_v, [rv, tvec])
            xx_tt = plsc.load_gather(
                xx_v, [jnp.full((16,), slot, jnp.int32), tvec])
            sqn_old = plsc.load_gather(sqn_v, [rv >> 4, rv & 15])
            omc = 1.0 - cvec
            sqn_new = (omc * omc * sqn_old + 2.0 * cvec * omc * drow
                       + cvec * cvec * xx_tt)
            plsc.store_scatter(sqn_v, [rv >> 4, rv & 15], sqn_new,
                               mask=validv)

            for j in range(4):
                r = lax.squeeze(lax.slice(rv, (j,), (j + 1,)), (0,))
                cr = lax.squeeze(lax.slice(cvec, (j,), (j + 1,)), (0,))
                rb_g = r * _D
                for cchunk in range(16):
                    xxc = xx_v[slot, pl.ds(cchunk * 16, 16)]
                    dsl = d_v[r, pl.ds(cchunk * 16, 16)]
                    d_v[r, pl.ds(cchunk * 16, 16)] = dsl + cr * (xxc - dsl)
                for jc in range(8):
                    xj = x_v[slot, pl.ds(jc * 16, 16)]
                    gs = g_v[pl.ds(rb_g + jc * 16, 16)]
                    g_v[pl.ds(rb_g + jc * 16, 16)] = gs + cr * (xj - gs)
            return carry

        lax.fori_loop(0, _STEPS, step, 0)
        pltpu.make_async_copy(xx_hbm.at[b, 0], xx_v.at[0], sem_xx).wait()
        pltpu.make_async_copy(emb_hbm.at[b, 0], x_v.at[0], sem_x).wait()

        def acc_row(r, accs):
            return tuple(accs[j] + g_v[pl.ds(r * _D + j * 16, 16)]
                         for j in range(8))
        accs = tuple(jnp.zeros((16,), jnp.float32) for _ in range(8))
        accs = lax.fori_loop(0, _N, acc_row, accs)
        for j in range(8):
            out_v[pl.ds(j * 16, 16)] = accs[j]
        pltpu.sync_copy(out_v, out_hbm.at[b])


def kernel(embeddings, nodes):
    nodes_flat = nodes.reshape(_N, _D)
    d0, xx, sqn0 = _gram(embeddings, nodes_flat)
    out = _som_sc(
        embeddings,
        nodes_flat.reshape(_N * _D),
        sqn0.reshape(16, 16),
        d0,
        xx,
    )
    return out
